# baseline (speedup 1.0000x reference)
import functools

import jax
import jax.numpy as jnp
from jax.experimental import pallas as pl
from jax.experimental.pallas import tpu as pltpu


def _fused_kernel(x_ref, adj_ref, w1_ref, b1_ref, w2_ref, b2_ref,
                  o_ref, s_ref, h_ref, *, ni):
    t = pl.program_id(0)

    @pl.when(t == 0)
    def _():
        s_ref[...] = jnp.dot(
            x_ref[...], w1_ref[...],
            preferred_element_type=jnp.float32).astype(jnp.bfloat16)

    @pl.when((t >= 1) & (t <= ni))
    def _():
        t1 = jnp.dot(adj_ref[...].astype(jnp.bfloat16), s_ref[...],
                     preferred_element_type=jnp.float32)
        bi = adj_ref.shape[0]
        h_ref[pl.ds((t - 1) * bi, bi), :] = jnp.maximum(
            t1 + b1_ref[...], 0.0).astype(jnp.bfloat16)

    @pl.when(t > ni)
    def _():
        g = jnp.dot(adj_ref[...].astype(jnp.bfloat16), h_ref[...],
                    preferred_element_type=jnp.float32)
        z = jnp.dot(g, w2_ref[...],
                    preferred_element_type=jnp.float32) + b2_ref[...]
        z = jnp.maximum(z, 0.0)
        o_ref[0, :, :] = jnp.sum(z, axis=0, keepdims=True)


def kernel(x, adj, W1, b1, W2, b2):
    batch, n, nfeat = x.shape
    nhid = W1.shape[1]
    x2 = x.reshape(n, nfeat)
    adj2 = adj.reshape(n, n)

    bi = 200
    ni = n // bi

    def adj_idx(t):
        return (jnp.where(t <= ni, jnp.maximum(t - 1, 0), 2 * ni - t), 0)

    def out_idx(t):
        return (jnp.clip(2 * ni - t, 0, ni - 1), 0, 0)

    partials = pl.pallas_call(
        functools.partial(_fused_kernel, ni=ni),
        grid=(1 + 2 * ni,),
        in_specs=[
            pl.BlockSpec((n, nfeat), lambda t: (0, 0)),
            pl.BlockSpec((bi, n), adj_idx),
            pl.BlockSpec((nfeat, nhid), lambda t: (0, 0)),
            pl.BlockSpec((1, nhid), lambda t: (0, 0)),
            pl.BlockSpec((nhid, nfeat), lambda t: (0, 0)),
            pl.BlockSpec((1, nfeat), lambda t: (0, 0)),
        ],
        out_specs=pl.BlockSpec((1, 1, nfeat), out_idx),
        out_shape=jax.ShapeDtypeStruct((ni, 1, nfeat), jnp.float32),
        scratch_shapes=[
            pltpu.VMEM((n, nhid), jnp.bfloat16),
            pltpu.VMEM((n, nhid), jnp.bfloat16),
        ],
        compiler_params=pltpu.CompilerParams(
            dimension_semantics=("arbitrary",)),
    )(x2, adj2, W1, b1.reshape(1, nhid), W2, b2.reshape(1, nfeat))

    return (jnp.sum(partials) / (n * nfeat)).reshape(batch)


# speedup vs baseline: 1.0334x; 1.0334x over previous
import functools

import jax
import jax.numpy as jnp
from jax.experimental import pallas as pl
from jax.experimental.pallas import tpu as pltpu


def _fused_kernel(x_ref, adj_ref, w1_ref, b1_ref, w2_ref, b2_ref,
                  o_ref, s_ref, h_ref, *, ni):
    t = pl.program_id(0)

    @pl.when(t == 0)
    def _():
        s_ref[...] = jnp.dot(
            x_ref[...].astype(jnp.bfloat16),
            w1_ref[...].astype(jnp.bfloat16),
            preferred_element_type=jnp.float32).astype(jnp.bfloat16)

    @pl.when(t < ni)
    def _():
        t1 = jnp.dot(adj_ref[...].astype(jnp.bfloat16), s_ref[...],
                     preferred_element_type=jnp.float32)
        bi = adj_ref.shape[0]
        h_ref[pl.ds(t * bi, bi), :] = jnp.maximum(
            t1 + b1_ref[...], 0.0).astype(jnp.bfloat16)

    @pl.when(t >= ni)
    def _():
        g = jnp.dot(adj_ref[...].astype(jnp.bfloat16), h_ref[...],
                    preferred_element_type=jnp.float32)
        z = jnp.dot(g, w2_ref[...],
                    preferred_element_type=jnp.float32) + b2_ref[...]
        z = jnp.maximum(z, 0.0)
        o_ref[0, :, :] = jnp.sum(z, axis=0, keepdims=True)


def kernel(x, adj, W1, b1, W2, b2):
    batch, n, nfeat = x.shape
    nhid = W1.shape[1]
    x2 = x.reshape(n, nfeat)
    adj2 = adj.reshape(n, n)

    bi = 400
    ni = n // bi

    def adj_idx(t):
        return (jnp.where(t < ni, t, 2 * ni - 1 - t), 0)

    def out_idx(t):
        return (jnp.clip(2 * ni - 1 - t, 0, ni - 1), 0, 0)

    partials = pl.pallas_call(
        functools.partial(_fused_kernel, ni=ni),
        grid=(2 * ni,),
        in_specs=[
            pl.BlockSpec((n, nfeat), lambda t: (0, 0)),
            pl.BlockSpec((bi, n), adj_idx),
            pl.BlockSpec((nfeat, nhid), lambda t: (0, 0)),
            pl.BlockSpec((1, nhid), lambda t: (0, 0)),
            pl.BlockSpec((nhid, nfeat), lambda t: (0, 0)),
            pl.BlockSpec((1, nfeat), lambda t: (0, 0)),
        ],
        out_specs=pl.BlockSpec((1, 1, nfeat), out_idx),
        out_shape=jax.ShapeDtypeStruct((ni, 1, nfeat), jnp.float32),
        scratch_shapes=[
            pltpu.VMEM((n, nhid), jnp.bfloat16),
            pltpu.VMEM((n, nhid), jnp.bfloat16),
        ],
        compiler_params=pltpu.CompilerParams(
            dimension_semantics=("arbitrary",)),
    )(x2, adj2, W1, b1.reshape(1, nhid), W2, b2.reshape(1, nfeat))

    return (jnp.sum(partials) / (n * nfeat)).reshape(batch)
